# baseline (speedup 1.0000x reference)
import functools

import jax
import jax.numpy as jnp
from jax import lax
from jax.experimental import pallas as pl
from jax.experimental.pallas import tpu as pltpu
from jax.experimental.pallas import tpu_sc as plsc

_B, _C, _IN = 128, 128, 500
_N = 512
_BBLK = 4

_NC, _NS, _L = 2, 16, 16
_NW = _NC * _NS
_BPW = _B // _NW
_G = 64
_NCHUNK = _N // _G


def _tc_body(x_ref, w_ref, b_ref, out_ref, rank_ref, val_ref, tmpl_ref):
    bias = b_ref[0, 0]
    for t in range(_BBLK):
        _tc_one(t, x_ref, w_ref, bias, out_ref, rank_ref, val_ref, tmpl_ref)


def _tc_one(t, x_ref, w_ref, bias, out_ref, rank_ref, val_ref, tmpl_ref):
    xb = x_ref[t]

    s500 = jnp.dot(w_ref[...], xb, preferred_element_type=jnp.float32) + bias
    s_row = jnp.concatenate(
        [s500, jnp.full((1, _N - _IN), bias, jnp.float32)], axis=1)

    s_mat = jnp.broadcast_to(s_row, (_N, _N))
    s_col_mat = s_mat.T

    gt = (s_mat > s_col_mat).astype(jnp.float32)
    eq = (s_mat == s_col_mat).astype(jnp.float32)
    a_idx = jax.lax.broadcasted_iota(jnp.int32, (_N, _N), 1).astype(jnp.float32)
    j_idx = jax.lax.broadcasted_iota(jnp.int32, (_N, _N), 0).astype(jnp.float32)
    lt_ij = (j_idx < a_idx).astype(jnp.float32)

    rank_c = jnp.sum(gt + eq * (a_idx < j_idx).astype(jnp.float32),
                     axis=1, keepdims=True)
    dupf_c = (jnp.sum(eq, axis=1, keepdims=True) >= 2.0).astype(jnp.float32)

    m = (rank_c == a_idx).astype(jnp.float32)
    is_dup_row = jnp.sum(m * dupf_c, axis=0, keepdims=True)
    nan_row = is_dup_row > 0.0
    nan = jnp.float32(jnp.nan)

    gt_r = jnp.sum((s_col_mat > s_mat).astype(jnp.float32), axis=0,
                   keepdims=True)
    tie_r = jnp.sum(eq * lt_ij, axis=0, keepdims=True)
    dup_r = jnp.sum(eq, axis=0, keepdims=True) >= 2.0

    rank_ref[t] = (gt_r + tie_r).astype(jnp.int32)
    val_ref[t] = jnp.where(dup_r, nan, 1.0)
    tmpl_ref[t] = jnp.where(nan_row, nan, 0.0)

    gathered = jnp.dot(xb, m[:_IN, :], preferred_element_type=jnp.float32)
    out_ref[t] = jnp.where(nan_row, nan, gathered)


def _tc_call(x, w2, b2):
    return pl.pallas_call(
        _tc_body,
        grid=(_B // _BBLK,),
        in_specs=[
            pl.BlockSpec((_BBLK, _C, _IN), lambda i: (i, 0, 0)),
            pl.BlockSpec((1, _C), lambda i: (0, 0)),
            pl.BlockSpec(memory_space=pltpu.SMEM),
        ],
        out_specs=[
            pl.BlockSpec((_BBLK, _C, _N), lambda i: (i, 0, 0)),
            pl.BlockSpec((_BBLK, 1, _N), lambda i: (i, 0, 0)),
            pl.BlockSpec((_BBLK, 1, _N), lambda i: (i, 0, 0)),
            pl.BlockSpec((_BBLK, 1, _N), lambda i: (i, 0, 0)),
        ],
        out_shape=[
            jax.ShapeDtypeStruct((_B, _C, _N), jnp.float32),
            jax.ShapeDtypeStruct((_B, 1, _N), jnp.int32),
            jax.ShapeDtypeStruct((_B, 1, _N), jnp.float32),
            jax.ShapeDtypeStruct((_B, 1, _N), jnp.float32),
        ],
    )(x, w2, b2)


def _sc_body(rank_hbm, val_hbm, tmpl_hbm, probs_hbm,
             rank_v, val_v, tmpl_v, buf0, buf1, sem0, sem1):
    wid = lax.axis_index("s") * _NC + lax.axis_index("c")
    bufs = (buf0, buf1)
    sems = (sem0, sem1)
    row_off = lax.iota(jnp.int32, _L) * _N

    for k in range(_BPW):
        b = wid * _BPW + k
        pltpu.sync_copy(rank_hbm.at[b], rank_v)
        pltpu.sync_copy(val_hbm.at[b], val_v)
        pltpu.sync_copy(tmpl_hbm.at[b], tmpl_v)

        tsegs = [tmpl_v[pl.ds(s * _L, _L)] for s in range(_N // _L)]
        for buf in bufs:
            def fill_row(r, carry):
                base = r * _N
                for s in range(_N // _L):
                    buf[pl.ds(base + s * _L, _L)] = tsegs[s]
                return carry
            lax.fori_loop(0, _G, fill_row, 0)

        def scatter(chunk, values_fn):
            j0 = chunk * _G
            for t in range(_G // _L):
                cols = rank_v[pl.ds(j0 + t * _L, _L)]
                vals = val_v[pl.ds(j0 + t * _L, _L)]
                plsc.store_scatter(bufs[chunk % 2],
                                   [row_off + t * (_L * _N) + cols],
                                   values_fn(vals))

        descs = [None, None]
        for c in range(_NCHUNK):
            i = c % 2
            if descs[i] is not None:
                descs[i].wait()
                scatter(c - 2, lambda v: jnp.where(v == 1.0, 0.0, v))
            scatter(c, lambda v: v)
            descs[i] = pltpu.async_copy(
                bufs[i], probs_hbm.at[b, pl.ds(c * _G * _N, _G * _N)], sems[i])
        for i in range(2):
            if descs[i] is not None:
                descs[i].wait()


_sc_call = functools.partial(
    pl.kernel,
    out_type=jax.ShapeDtypeStruct((_B, _N * _N), jnp.float32),
    mesh=plsc.VectorSubcoreMesh(core_axis_name="c", subcore_axis_name="s"),
    compiler_params=pltpu.CompilerParams(needs_layout_passes=False),
    scratch_types=[
        pltpu.VMEM((_N,), jnp.int32),
        pltpu.VMEM((_N,), jnp.float32),
        pltpu.VMEM((_N,), jnp.float32),
        pltpu.VMEM((_G * _N,), jnp.float32),
        pltpu.VMEM((_G * _N,), jnp.float32),
        pltpu.SemaphoreType.DMA,
        pltpu.SemaphoreType.DMA,
    ],
)(_sc_body)


def kernel(x, W, b):
    w2 = W.reshape(1, _C)
    b2 = jnp.reshape(b, (1, 1)).astype(jnp.float32)
    out, rank, val, tmpl = _tc_call(x, w2, b2)
    probs_t = _sc_call(rank.reshape(_B, _N), val.reshape(_B, _N),
                       tmpl.reshape(_B, _N))
    return out, probs_t.reshape(_B, _N, _N)


# speedup vs baseline: 1.4126x; 1.4126x over previous
import functools

import jax
import jax.numpy as jnp
from jax import lax
from jax.experimental import pallas as pl
from jax.experimental.pallas import tpu as pltpu
from jax.experimental.pallas import tpu_sc as plsc

_B, _C, _IN = 128, 128, 500
_N = 512
_BBLK = 4

_NC, _NS, _L = 2, 16, 16
_NW = _NC * _NS
_BPW = _B // _NW
_G = 64
_NCHUNK = _N // _G


def _tc_body(x_ref, w_ref, b_ref, out_ref, rank_ref, val_ref, tmpl_ref):
    bias = b_ref[0, 0]
    for t in range(_BBLK):
        _tc_one(t, x_ref, w_ref, bias, out_ref, rank_ref, val_ref, tmpl_ref)


def _tc_one(t, x_ref, w_ref, bias, out_ref, rank_ref, val_ref, tmpl_ref):
    xb = x_ref[t]

    s500 = jnp.dot(w_ref[...], xb, preferred_element_type=jnp.float32) + bias
    s_row = jnp.concatenate(
        [s500, jnp.full((1, _N - _IN), bias, jnp.float32)], axis=1)

    s_mat = jnp.broadcast_to(s_row, (_N, _N))
    s_col_mat = s_mat.T

    gt = (s_mat > s_col_mat).astype(jnp.float32)
    eq = (s_mat == s_col_mat).astype(jnp.float32)
    a_idx = jax.lax.broadcasted_iota(jnp.int32, (_N, _N), 1).astype(jnp.float32)
    j_idx = jax.lax.broadcasted_iota(jnp.int32, (_N, _N), 0).astype(jnp.float32)
    lt_ij = (j_idx < a_idx).astype(jnp.float32)

    rank_c = jnp.sum(gt + eq * (a_idx < j_idx).astype(jnp.float32),
                     axis=1, keepdims=True)
    dupf_c = (jnp.sum(eq, axis=1, keepdims=True) >= 2.0).astype(jnp.float32)

    m = (rank_c == a_idx).astype(jnp.float32)
    is_dup_row = jnp.sum(m * dupf_c, axis=0, keepdims=True)
    nan_row = is_dup_row > 0.0
    nan = jnp.float32(jnp.nan)

    gt_r = jnp.sum((s_col_mat > s_mat).astype(jnp.float32), axis=0,
                   keepdims=True)
    tie_r = jnp.sum(eq * lt_ij, axis=0, keepdims=True)
    dup_r = jnp.sum(eq, axis=0, keepdims=True) >= 2.0

    rank_ref[t] = (gt_r + tie_r).astype(jnp.int32)
    val_ref[t] = jnp.where(dup_r, nan, 1.0)
    tmpl_ref[t] = jnp.where(nan_row, nan, 0.0)

    gathered = jnp.dot(xb, m[:_IN, :], preferred_element_type=jnp.float32)
    out_ref[t] = jnp.where(nan_row, nan, gathered)


def _tc_call(x, w2, b2):
    return pl.pallas_call(
        _tc_body,
        grid=(_B // _BBLK,),
        in_specs=[
            pl.BlockSpec((_BBLK, _C, _IN), lambda i: (i, 0, 0)),
            pl.BlockSpec((1, _C), lambda i: (0, 0)),
            pl.BlockSpec(memory_space=pltpu.SMEM),
        ],
        out_specs=[
            pl.BlockSpec((_BBLK, _C, _N), lambda i: (i, 0, 0)),
            pl.BlockSpec((_BBLK, 1, _N), lambda i: (i, 0, 0)),
            pl.BlockSpec((_BBLK, 1, _N), lambda i: (i, 0, 0)),
            pl.BlockSpec((_BBLK, 1, _N), lambda i: (i, 0, 0)),
        ],
        out_shape=[
            jax.ShapeDtypeStruct((_B, _C, _N), jnp.float32),
            jax.ShapeDtypeStruct((_B, 1, _N), jnp.int32),
            jax.ShapeDtypeStruct((_B, 1, _N), jnp.float32),
            jax.ShapeDtypeStruct((_B, 1, _N), jnp.float32),
        ],
    )(x, w2, b2)


def _sc_body(rank_hbm, val_hbm, tmpl_hbm, probs_hbm,
             rank_v, val_v, tmpl_v, buf0, buf1, sem0, sem1):
    wid = lax.axis_index("s") * _NC + lax.axis_index("c")
    bufs = (buf0, buf1)
    sems = (sem0, sem1)
    row_iota = lax.iota(jnp.int32, _L)

    for k in range(_BPW):
        b = wid * _BPW + k
        pltpu.sync_copy(rank_hbm.at[b], rank_v)
        pltpu.sync_copy(val_hbm.at[b], val_v)
        pltpu.sync_copy(tmpl_hbm.at[b], tmpl_v)

        tsegs = [tmpl_v[pl.ds(s * _L, _L)] for s in range(_N // _L)]
        for buf in bufs:
            def fill_row(r, carry):
                for s in range(_N // _L):
                    buf[r, pl.ds(s * _L, _L)] = tsegs[s]
                return carry
            lax.fori_loop(0, _G, fill_row, 0)

        def scatter(chunk, values_fn):
            j0 = chunk * _G
            for t in range(_G // _L):
                cols = rank_v[pl.ds(j0 + t * _L, _L)]
                vals = val_v[pl.ds(j0 + t * _L, _L)]
                plsc.store_scatter(bufs[chunk % 2],
                                   [row_iota + t * _L, cols],
                                   values_fn(vals))

        descs = [None, None]
        for c in range(_NCHUNK):
            i = c % 2
            if descs[i] is not None:
                descs[i].wait()
                scatter(c - 2, lambda v: jnp.where(v == 1.0, 0.0, v))
            scatter(c, lambda v: v)
            descs[i] = pltpu.async_copy(
                bufs[i], probs_hbm.at[b, pl.ds(c * _G, _G), :], sems[i])
        for i in range(2):
            if descs[i] is not None:
                descs[i].wait()


_sc_call = functools.partial(
    pl.kernel,
    out_type=jax.ShapeDtypeStruct((_B, _N, _N), jnp.float32),
    mesh=plsc.VectorSubcoreMesh(core_axis_name="c", subcore_axis_name="s"),
    compiler_params=pltpu.CompilerParams(needs_layout_passes=False),
    scratch_types=[
        pltpu.VMEM((_N,), jnp.int32),
        pltpu.VMEM((_N,), jnp.float32),
        pltpu.VMEM((_N,), jnp.float32),
        pltpu.VMEM((_G, _N), jnp.float32),
        pltpu.VMEM((_G, _N), jnp.float32),
        pltpu.SemaphoreType.DMA,
        pltpu.SemaphoreType.DMA,
    ],
)(_sc_body)


def kernel(x, W, b):
    w2 = W.reshape(1, _C)
    b2 = jnp.reshape(b, (1, 1)).astype(jnp.float32)
    out, rank, val, tmpl = _tc_call(x, w2, b2)
    probs_t = _sc_call(rank.reshape(_B, _N), val.reshape(_B, _N),
                       tmpl.reshape(_B, _N))
    return out, probs_t


# speedup vs baseline: 2.3929x; 1.6939x over previous
import jax
import jax.numpy as jnp
from jax.experimental import pallas as pl
from jax.experimental.pallas import tpu as pltpu

_B, _C, _IN = 128, 128, 500
_N = 512


_BBLK = 4


def _body(x_ref, w_ref, b_ref, out_ref, probs_t_ref):
    bias = b_ref[0, 0]
    for t in range(_BBLK):
        _one_batch(t, x_ref, w_ref, bias, out_ref, probs_t_ref)


def _one_batch(t, x_ref, w_ref, bias, out_ref, probs_t_ref):
    xb = x_ref[t]

    s500 = jnp.dot(w_ref[...], xb, preferred_element_type=jnp.float32) + bias
    s_row = jnp.concatenate(
        [s500, jnp.full((1, _N - _IN), bias, jnp.float32)], axis=1)

    s_mat = jnp.broadcast_to(s_row, (_N, _N))
    s_col_mat = s_mat.T

    gt = (s_mat > s_col_mat).astype(jnp.float32)
    eq = (s_mat == s_col_mat).astype(jnp.float32)
    a_idx = jax.lax.broadcasted_iota(jnp.int32, (_N, _N), 1).astype(jnp.float32)
    j_idx = jax.lax.broadcasted_iota(jnp.int32, (_N, _N), 0).astype(jnp.float32)
    tie = eq * (a_idx < j_idx).astype(jnp.float32)

    rank = jnp.sum(gt + tie, axis=1, keepdims=True)
    dupf = (jnp.sum(eq, axis=1, keepdims=True) >= 2.0).astype(jnp.float32)

    m = (rank == a_idx).astype(jnp.float32)

    is_dup_row = jnp.sum(m * dupf, axis=0, keepdims=True)
    nan_row = is_dup_row > 0.0

    nan = jnp.float32(jnp.nan)
    probs_t_ref[t] = jnp.where(nan_row, nan, m)

    gathered = jnp.dot(xb, m[:_IN, :], preferred_element_type=jnp.float32)
    out_ref[t] = jnp.where(nan_row, nan, gathered)


def kernel(x, W, b):
    w2 = W.reshape(1, _C)
    b2 = jnp.reshape(b, (1, 1)).astype(jnp.float32)
    out, probs_t = pl.pallas_call(
        _body,
        grid=(_B // _BBLK,),
        in_specs=[
            pl.BlockSpec((_BBLK, _C, _IN), lambda i: (i, 0, 0)),
            pl.BlockSpec((1, _C), lambda i: (0, 0)),
            pl.BlockSpec(memory_space=pltpu.SMEM),
        ],
        out_specs=[
            pl.BlockSpec((_BBLK, _C, _N), lambda i: (i, 0, 0)),
            pl.BlockSpec((_BBLK, _N, _N), lambda i: (i, 0, 0)),
        ],
        out_shape=[
            jax.ShapeDtypeStruct((_B, _C, _N), jnp.float32),
            jax.ShapeDtypeStruct((_B, _N, _N), jnp.float32),
        ],
    )(x, w2, b2)
    return out, probs_t
